# baseline (speedup 1.0000x reference)
import jax
import jax.numpy as jnp
from jax.experimental import pallas as pl
from jax.experimental.pallas import tpu as pltpu
from jax.experimental.pallas import tpu_sc as plsc

WINDOW = 128


def kernel(x, weight):
    batch = x.shape[0]
    num_rows, embed_dim = weight.shape
    wpair = weight.reshape(num_rows // 2, 2 * embed_dim)
    pair_idx = jax.lax.shift_right_logical(x, 1).reshape(1, batch)

    mesh = plsc.VectorSubcoreMesh(
        core_axis_name="core", subcore_axis_name="subcore"
    )

    @pl.kernel(
        out_type=jax.ShapeDtypeStruct((batch, 2 * embed_dim), weight.dtype),
        mesh=mesh,
        compiler_params=pltpu.CompilerParams(use_tc_tiling_on_sc=False),
    )
    def gather_kernel(w_hbm, i_hbm, o_hbm):
        def body(i_vmem, o_vmem):
            pltpu.sync_copy(w_hbm.at[i_vmem.at[0]], o_vmem)

        pltpu.emit_pipeline(
            body,
            grid=(batch // WINDOW,),
            in_specs=[pl.BlockSpec((1, WINDOW), lambda i: (0, i))],
            out_specs=[
                pl.BlockSpec((WINDOW, 2 * embed_dim), lambda i: (i, 0))
            ],
            core_axis_name=("core", "subcore"),
            dimension_semantics=(pltpu.PARALLEL,),
        )(i_hbm, o_hbm)

    pairs = gather_kernel(wpair, pair_idx)
    odd = jnp.bitwise_and(x, 1)[:, None] == 1
    return jnp.where(odd, pairs[:, embed_dim:], pairs[:, :embed_dim])


# speedup vs baseline: 1.2410x; 1.2410x over previous
import jax
import jax.numpy as jnp
from jax import lax
from jax.experimental import pallas as pl
from jax.experimental.pallas import tpu as pltpu
from jax.experimental.pallas import tpu_sc as plsc

LANES = 16
NUM_WORKERS = 32
PANEL_W = 16384
POS_BITS = 15


def kernel(x, weight):
    batch = x.shape[0]
    num_rows, embed_dim = weight.shape
    n_buckets = (num_rows + PANEL_W - 1) // PANEL_W
    enc_len = batch + n_buckets * LANES
    out_pad_w = batch + LANES

    wt = weight.T

    b = lax.shift_right_logical(x, 14)
    local = jnp.bitwise_and(x, PANEL_W - 1)
    onehot = (b[:, None] == jnp.arange(n_buckets, dtype=jnp.int32)).astype(
        jnp.int32
    )
    cnt = jnp.sum(onehot, axis=0)
    pcnt = ((cnt + LANES - 1) // LANES) * LANES
    csum = jnp.cumsum(pcnt)
    base = csum - pcnt
    group_bounds = jnp.concatenate([base // LANES, csum // LANES]).astype(
        jnp.int32
    )
    rank = jnp.take_along_axis(
        jnp.cumsum(onehot, axis=0) - onehot, b[:, None], axis=1
    )[:, 0]
    slot = base[b] + rank
    enc_vals = jnp.bitwise_or(
        lax.shift_left(jnp.arange(batch, dtype=jnp.int32), POS_BITS), local
    )
    prefill = lax.shift_left(
        batch + jnp.bitwise_and(jnp.arange(enc_len, dtype=jnp.int32), LANES - 1),
        POS_BITS,
    )
    enc = prefill.at[slot].set(enc_vals)

    mesh = plsc.VectorSubcoreMesh(
        core_axis_name="core", subcore_axis_name="subcore"
    )
    last_start = (n_buckets - 1) * PANEL_W
    last_full = ((num_rows - last_start) // 128) * 128
    tail_w = num_rows - last_start - last_full
    wtail = jnp.pad(
        weight[num_rows - tail_w :].T, ((0, 0), (0, 128 - tail_w))
    )

    @pl.kernel(
        out_type=jax.ShapeDtypeStruct((embed_dim, batch), weight.dtype),
        mesh=mesh,
        compiler_params=pltpu.CompilerParams(needs_layout_passes=False),
        scratch_types=[
            pltpu.VMEM((2, PANEL_W), jnp.float32),
            pltpu.VMEM((2, PANEL_W), jnp.float32),
            pltpu.VMEM((2, out_pad_w), jnp.float32),
            pltpu.VMEM((enc_len,), jnp.int32),
            pltpu.VMEM((2 * n_buckets,), jnp.int32),
            pltpu.SemaphoreType.DMA,
            pltpu.SemaphoreType.DMA,
            pltpu.SemaphoreType.DMA,
            pltpu.SemaphoreType.DMA,
        ],
    )
    def gather_kernel(
        wt_hbm, wtail_hbm, enc_hbm, bounds_hbm, ot_hbm,
        panel_a, panel_b, out_v, enc_v, bounds_v,
        sem_e, sem_s, sem_a, sem_b,
    ):
        w = lax.axis_index("subcore") * 2 + lax.axis_index("core")
        j0 = 2 * w

        h_enc = pltpu.async_copy(enc_hbm, enc_v, sem_e)
        pltpu.sync_copy(bounds_hbm, bounds_v)
        lane = lax.iota(jnp.int32, 16)
        starts = bounds_v[pl.ds(0, LANES)]
        ends = bounds_v[pl.ds(n_buckets, LANES)]

        bufs = (panel_a, panel_b)
        sems = (sem_a, sem_b)

        def issue_panel(bk, buf, sem):
            start = bk * PANEL_W
            if bk < n_buckets - 1:
                return [
                    pltpu.async_copy(
                        wt_hbm.at[pl.ds(j0, 2), pl.ds(start, PANEL_W)],
                        buf,
                        sem,
                    )
                ]
            hs = [
                pltpu.async_copy(
                    wt_hbm.at[pl.ds(j0, 2), pl.ds(start, last_full)],
                    buf.at[:, pl.ds(0, last_full)],
                    sem,
                )
            ]
            if tail_w:
                hs.append(
                    pltpu.async_copy(
                        wtail_hbm.at[pl.ds(j0, 2), :],
                        buf.at[:, pl.ds(last_full, 128)],
                        sem,
                    )
                )
            return hs

        handles = [issue_panel(0, bufs[0], sems[0]), None]
        h_enc.wait()

        for bk in range(n_buckets):
            cur = bufs[bk & 1]
            for h in handles[bk & 1]:
                h.wait()
            if bk < n_buckets - 1:
                handles[(bk + 1) & 1] = issue_panel(
                    bk + 1, bufs[(bk + 1) & 1], sems[(bk + 1) & 1]
                )
            sel = (lane == bk).astype(jnp.int32)
            gs = jnp.sum(sel * starts, axis=0)
            ge = jnp.sum(sel * ends, axis=0)

            row0 = jnp.zeros((LANES,), jnp.int32)
            row1 = jnp.ones((LANES,), jnp.int32)

            @pl.loop(gs, ge)
            def _(g):
                e = enc_v[pl.ds(g * LANES, LANES)]
                loc = jnp.bitwise_and(e, PANEL_W - 1)
                pos = lax.shift_right_logical(e, POS_BITS)
                v0 = plsc.load_gather(cur, [row0, loc])
                v1 = plsc.load_gather(cur, [row1, loc])
                plsc.store_scatter(out_v, [row0, pos], v0)
                plsc.store_scatter(out_v, [row1, pos], v1)

        h0 = pltpu.async_copy(
            out_v.at[0, pl.ds(0, batch)], ot_hbm.at[j0], sem_a
        )
        h1 = pltpu.async_copy(
            out_v.at[1, pl.ds(0, batch)], ot_hbm.at[j0 + 1], sem_b
        )
        h0.wait()
        h1.wait()

    ot = gather_kernel(wt, wtail, enc, group_bounds)
    return ot.T


# speedup vs baseline: 2.7601x; 2.2240x over previous
import jax
import jax.numpy as jnp
from jax import lax
from jax.experimental import pallas as pl
from jax.experimental.pallas import tpu as pltpu
from jax.experimental.pallas import tpu_sc as plsc

LANES = 16
NUM_WORKERS = 32
PANEL_W = 16384
MASK14 = PANEL_W - 1


def kernel(x, weight):
    batch = x.shape[0]
    num_rows, embed_dim = weight.shape
    n_buckets = (num_rows + PANEL_W - 1) // PANEL_W
    n_groups = batch // LANES

    wt = weight.T

    xu = x.astype(jnp.uint32)
    pos = jnp.arange(batch, dtype=jnp.uint32)
    key = (
        lax.shift_left(lax.shift_right_logical(xu, jnp.uint32(14)), jnp.uint32(28))
        | lax.shift_left(pos, jnp.uint32(14))
        | (xu & jnp.uint32(MASK14))
    )
    key = jnp.sort(key)
    starts = jnp.searchsorted(
        key,
        lax.shift_left(
            jnp.arange(n_buckets, dtype=jnp.uint32), jnp.uint32(28)
        ),
    ).astype(jnp.int32)
    enc = lax.bitcast_convert_type(key, jnp.int32)
    ends = jnp.concatenate(
        [starts[1:], jnp.full((1,), batch, jnp.int32)]
    )
    group_bounds = jnp.concatenate([starts, ends])

    mesh = plsc.VectorSubcoreMesh(
        core_axis_name="core", subcore_axis_name="subcore"
    )
    last_start = (n_buckets - 1) * PANEL_W
    last_full = ((num_rows - last_start) // 128) * 128
    tail_w = num_rows - last_start - last_full
    wtail = jnp.pad(
        weight[num_rows - tail_w :].T, ((0, 0), (0, 128 - tail_w))
    )

    @pl.kernel(
        out_type=jax.ShapeDtypeStruct((embed_dim, batch), weight.dtype),
        mesh=mesh,
        compiler_params=pltpu.CompilerParams(needs_layout_passes=False),
        scratch_types=[
            pltpu.VMEM((2, PANEL_W), jnp.float32),
            pltpu.VMEM((2, PANEL_W), jnp.float32),
            pltpu.VMEM((2, batch), jnp.float32),
            pltpu.VMEM((batch,), jnp.int32),
            pltpu.VMEM((2 * n_buckets,), jnp.int32),
            pltpu.SemaphoreType.DMA,
            pltpu.SemaphoreType.DMA,
            pltpu.SemaphoreType.DMA,
            pltpu.SemaphoreType.DMA,
        ],
    )
    def gather_kernel(
        wt_hbm, wtail_hbm, enc_hbm, bounds_hbm, ot_hbm,
        panel_a, panel_b, out_v, enc_v, bounds_v,
        sem_e, sem_s, sem_a, sem_b,
    ):
        w = lax.axis_index("subcore") * 2 + lax.axis_index("core")
        j0 = 2 * w

        h_enc = pltpu.async_copy(enc_hbm, enc_v, sem_e)
        pltpu.sync_copy(bounds_hbm, bounds_v)
        lane = lax.iota(jnp.int32, LANES)
        starts_v = bounds_v[pl.ds(0, LANES)]
        ends_v = bounds_v[pl.ds(n_buckets, LANES)]

        bufs = (panel_a, panel_b)
        sems = (sem_a, sem_b)

        def issue_panel(bk, buf, sem):
            start = bk * PANEL_W
            if bk < n_buckets - 1:
                return [
                    pltpu.async_copy(
                        wt_hbm.at[pl.ds(j0, 2), pl.ds(start, PANEL_W)],
                        buf,
                        sem,
                    )
                ]
            hs = [
                pltpu.async_copy(
                    wt_hbm.at[pl.ds(j0, 2), pl.ds(start, last_full)],
                    buf.at[:, pl.ds(0, last_full)],
                    sem,
                )
            ]
            if tail_w:
                hs.append(
                    pltpu.async_copy(
                        wtail_hbm.at[pl.ds(j0, 2), :],
                        buf.at[:, pl.ds(last_full, 128)],
                        sem,
                    )
                )
            return hs

        handles = [issue_panel(0, bufs[0], sems[0]), None]
        h_enc.wait()

        row0 = jnp.zeros((LANES,), jnp.int32)
        row1 = jnp.ones((LANES,), jnp.int32)

        for bk in range(n_buckets):
            cur = bufs[bk & 1]
            for h in handles[bk & 1]:
                h.wait()
            if bk < n_buckets - 1:
                handles[(bk + 1) & 1] = issue_panel(
                    bk + 1, bufs[(bk + 1) & 1], sems[(bk + 1) & 1]
                )
            sel = (lane == bk).astype(jnp.int32)
            s_slot = jnp.sum(sel * starts_v, axis=0)
            e_slot = jnp.sum(sel * ends_v, axis=0)
            gs = lax.shift_right_logical(s_slot, 4)
            ge = lax.shift_right_logical(e_slot + LANES - 1, 4)

            @pl.loop(gs, ge)
            def _(g):
                k = g * LANES + lane
                m = jnp.logical_and(k >= s_slot, k < e_slot)
                e = enc_v[pl.ds(g * LANES, LANES)]
                loc = jnp.bitwise_and(e, MASK14)
                p = jnp.bitwise_and(
                    lax.shift_right_logical(e, 14), MASK14
                )
                v0 = plsc.load_gather(cur, [row0, loc])
                v1 = plsc.load_gather(cur, [row1, loc])
                plsc.store_scatter(out_v, [row0, p], v0, mask=m)
                plsc.store_scatter(out_v, [row1, p], v1, mask=m)

        h0 = pltpu.async_copy(out_v.at[0], ot_hbm.at[j0], sem_a)
        h1 = pltpu.async_copy(out_v.at[1], ot_hbm.at[j0 + 1], sem_b)
        h0.wait()
        h1.wait()

    ot = gather_kernel(wt, wtail, enc, group_bounds)
    return ot.T


# speedup vs baseline: 2.7601x; 1.0000x over previous
import jax
import jax.numpy as jnp
from jax import lax
from jax.experimental import pallas as pl
from jax.experimental.pallas import tpu as pltpu
from jax.experimental.pallas import tpu_sc as plsc

LANES = 16
NUM_WORKERS = 32
PANEL_W = 16384
MASK14 = PANEL_W - 1


def kernel(x, weight):
    batch = x.shape[0]
    num_rows, embed_dim = weight.shape
    n_buckets = (num_rows + PANEL_W - 1) // PANEL_W
    n_groups = batch // LANES

    wt = weight.T

    xu = x.astype(jnp.uint32)
    pos = jnp.arange(batch, dtype=jnp.uint32)
    key = (
        lax.shift_left(lax.shift_right_logical(xu, jnp.uint32(14)), jnp.uint32(28))
        | lax.shift_left(pos, jnp.uint32(14))
        | (xu & jnp.uint32(MASK14))
    )
    key = jnp.sort(key)
    starts = jnp.searchsorted(
        key,
        lax.shift_left(
            jnp.arange(n_buckets, dtype=jnp.uint32), jnp.uint32(28)
        ),
    ).astype(jnp.int32)
    enc = lax.bitcast_convert_type(key, jnp.int32)
    ends = jnp.concatenate(
        [starts[1:], jnp.full((1,), batch, jnp.int32)]
    )
    group_bounds = jnp.concatenate([starts, ends])

    mesh = plsc.VectorSubcoreMesh(
        core_axis_name="core", subcore_axis_name="subcore"
    )
    last_start = (n_buckets - 1) * PANEL_W
    last_full = ((num_rows - last_start) // 128) * 128
    tail_w = num_rows - last_start - last_full
    wtail = jnp.pad(
        weight[num_rows - tail_w :].T, ((0, 0), (0, 128 - tail_w))
    )

    @pl.kernel(
        out_type=jax.ShapeDtypeStruct((embed_dim, batch), weight.dtype),
        mesh=mesh,
        compiler_params=pltpu.CompilerParams(needs_layout_passes=False),
        scratch_types=[
            pltpu.VMEM((2, PANEL_W), jnp.float32),
            pltpu.VMEM((2, PANEL_W), jnp.float32),
            pltpu.VMEM((2, batch), jnp.float32),
            pltpu.VMEM((batch,), jnp.int32),
            pltpu.VMEM((2 * n_buckets,), jnp.int32),
            pltpu.SemaphoreType.DMA,
            pltpu.SemaphoreType.DMA,
            pltpu.SemaphoreType.DMA,
            pltpu.SemaphoreType.DMA,
        ],
    )
    def gather_kernel(
        wt_hbm, wtail_hbm, enc_hbm, bounds_hbm, ot_hbm,
        panel_a, panel_b, out_v, enc_v, bounds_v,
        sem_e, sem_s, sem_a, sem_b,
    ):
        w = lax.axis_index("subcore") * 2 + lax.axis_index("core")
        j0 = 2 * w

        h_enc = pltpu.async_copy(enc_hbm, enc_v, sem_e)
        pltpu.sync_copy(bounds_hbm, bounds_v)
        lane = lax.iota(jnp.int32, LANES)
        starts_v = bounds_v[pl.ds(0, LANES)]
        ends_v = bounds_v[pl.ds(n_buckets, LANES)]

        bufs = (panel_a, panel_b)
        sems = (sem_a, sem_b)

        def issue_panel(bk, buf, sem):
            start = bk * PANEL_W
            if bk < n_buckets - 1:
                return [
                    pltpu.async_copy(
                        wt_hbm.at[pl.ds(j0, 2), pl.ds(start, PANEL_W)],
                        buf,
                        sem,
                    )
                ]
            hs = [
                pltpu.async_copy(
                    wt_hbm.at[pl.ds(j0, 2), pl.ds(start, last_full)],
                    buf.at[:, pl.ds(0, last_full)],
                    sem,
                )
            ]
            if tail_w:
                hs.append(
                    pltpu.async_copy(
                        wtail_hbm.at[pl.ds(j0, 2), :],
                        buf.at[:, pl.ds(last_full, 128)],
                        sem,
                    )
                )
            return hs

        handles = [issue_panel(0, bufs[0], sems[0]), None]
        h_enc.wait()

        row0 = jnp.zeros((LANES,), jnp.int32)
        row1 = jnp.ones((LANES,), jnp.int32)

        for bk in range(n_buckets):
            cur = bufs[bk & 1]
            for h in handles[bk & 1]:
                h.wait()
            if bk < n_buckets - 1:
                handles[(bk + 1) & 1] = issue_panel(
                    bk + 1, bufs[(bk + 1) & 1], sems[(bk + 1) & 1]
                )
            sel = (lane == bk).astype(jnp.int32)
            s_slot = jnp.sum(sel * starts_v, axis=0)
            e_slot = jnp.sum(sel * ends_v, axis=0)
            gs = lax.shift_right_logical(s_slot, 4)
            ge = lax.shift_right_logical(e_slot + LANES - 1, 4)

            def do_group(g, masked):
                e = enc_v[pl.ds(g * LANES, LANES)]
                loc = jnp.bitwise_and(e, MASK14)
                p = jnp.bitwise_and(
                    lax.shift_right_logical(e, 14), MASK14
                )
                v0 = plsc.load_gather(cur, [row0, loc])
                v1 = plsc.load_gather(cur, [row1, loc])
                if masked:
                    k = g * LANES + lane
                    m = jnp.logical_and(k >= s_slot, k < e_slot)
                    plsc.store_scatter(out_v, [row0, p], v0, mask=m)
                    plsc.store_scatter(out_v, [row1, p], v1, mask=m)
                else:
                    plsc.store_scatter(out_v, [row0, p], v0)
                    plsc.store_scatter(out_v, [row1, p], v1)

            @pl.when(gs < ge)
            def _():
                do_group(gs, True)

            @pl.when(ge > gs + 1)
            def _():
                do_group(ge - 1, True)

            @pl.loop(gs + 1, ge - 1)
            def _(g):
                do_group(g, False)

        h0 = pltpu.async_copy(out_v.at[0], ot_hbm.at[j0], sem_a)
        h1 = pltpu.async_copy(out_v.at[1], ot_hbm.at[j0 + 1], sem_b)
        h0.wait()
        h1.wait()

    ot = gather_kernel(wt, wtail, enc, group_bounds)
    return ot.T


# speedup vs baseline: 2.8778x; 1.0427x over previous
import jax
import jax.numpy as jnp
from jax import lax
from jax.experimental import pallas as pl
from jax.experimental.pallas import tpu as pltpu
from jax.experimental.pallas import tpu_sc as plsc

LANES = 16
NUM_WORKERS = 32
PANEL_W = 16384
MASK14 = PANEL_W - 1


def kernel(x, weight):
    batch = x.shape[0]
    num_rows, embed_dim = weight.shape
    n_buckets = (num_rows + PANEL_W - 1) // PANEL_W
    n_groups = batch // LANES

    wt = weight.T

    xu = x.astype(jnp.uint32)
    pos = jnp.arange(batch, dtype=jnp.uint32)
    key = (
        lax.shift_left(lax.shift_right_logical(xu, jnp.uint32(14)), jnp.uint32(28))
        | lax.shift_left(pos, jnp.uint32(14))
        | (xu & jnp.uint32(MASK14))
    )
    key = jnp.sort(key)
    enc = lax.bitcast_convert_type(key, jnp.int32)
    b = lax.shift_right_logical(x, 14)
    cnt = jnp.sum(
        (b[:, None] == jnp.arange(n_buckets, dtype=jnp.int32)).astype(
            jnp.int32
        ),
        axis=0,
    )
    ends = jnp.cumsum(cnt)
    starts = ends - cnt
    group_bounds = jnp.concatenate([starts, ends])

    mesh = plsc.VectorSubcoreMesh(
        core_axis_name="core", subcore_axis_name="subcore"
    )
    last_start = (n_buckets - 1) * PANEL_W
    last_full = ((num_rows - last_start) // 128) * 128
    tail_w = num_rows - last_start - last_full
    wtail = jnp.pad(
        weight[num_rows - tail_w :].T, ((0, 0), (0, 128 - tail_w))
    )

    @pl.kernel(
        out_type=jax.ShapeDtypeStruct((embed_dim, batch), weight.dtype),
        mesh=mesh,
        compiler_params=pltpu.CompilerParams(needs_layout_passes=False),
        scratch_types=[
            pltpu.VMEM((2, PANEL_W), jnp.float32),
            pltpu.VMEM((2, PANEL_W), jnp.float32),
            pltpu.VMEM((2, batch), jnp.float32),
            pltpu.VMEM((batch,), jnp.int32),
            pltpu.VMEM((2 * n_buckets,), jnp.int32),
            pltpu.SemaphoreType.DMA,
            pltpu.SemaphoreType.DMA,
            pltpu.SemaphoreType.DMA,
            pltpu.SemaphoreType.DMA,
        ],
    )
    def gather_kernel(
        wt_hbm, wtail_hbm, enc_hbm, bounds_hbm, ot_hbm,
        panel_a, panel_b, out_v, enc_v, bounds_v,
        sem_e, sem_s, sem_a, sem_b,
    ):
        w = lax.axis_index("subcore") * 2 + lax.axis_index("core")
        j0 = 2 * w

        h_enc = pltpu.async_copy(enc_hbm, enc_v, sem_e)
        pltpu.sync_copy(bounds_hbm, bounds_v)
        lane = lax.iota(jnp.int32, LANES)
        starts_v = bounds_v[pl.ds(0, LANES)]
        ends_v = bounds_v[pl.ds(n_buckets, LANES)]

        bufs = (panel_a, panel_b)
        sems = (sem_a, sem_b)

        def issue_panel(bk, buf, sem):
            start = bk * PANEL_W
            if bk < n_buckets - 1:
                return [
                    pltpu.async_copy(
                        wt_hbm.at[pl.ds(j0, 2), pl.ds(start, PANEL_W)],
                        buf,
                        sem,
                    )
                ]
            hs = [
                pltpu.async_copy(
                    wt_hbm.at[pl.ds(j0, 2), pl.ds(start, last_full)],
                    buf.at[:, pl.ds(0, last_full)],
                    sem,
                )
            ]
            if tail_w:
                hs.append(
                    pltpu.async_copy(
                        wtail_hbm.at[pl.ds(j0, 2), :],
                        buf.at[:, pl.ds(last_full, 128)],
                        sem,
                    )
                )
            return hs

        handles = [issue_panel(0, bufs[0], sems[0]), None]
        h_enc.wait()

        row0 = jnp.zeros((LANES,), jnp.int32)
        row1 = jnp.ones((LANES,), jnp.int32)

        for bk in range(n_buckets):
            cur = bufs[bk & 1]
            for h in handles[bk & 1]:
                h.wait()
            if bk < n_buckets - 1:
                handles[(bk + 1) & 1] = issue_panel(
                    bk + 1, bufs[(bk + 1) & 1], sems[(bk + 1) & 1]
                )
            sel = (lane == bk).astype(jnp.int32)
            s_slot = jnp.sum(sel * starts_v, axis=0)
            e_slot = jnp.sum(sel * ends_v, axis=0)
            gs = lax.shift_right_logical(s_slot, 4)
            ge = lax.shift_right_logical(e_slot + LANES - 1, 4)

            def do_group(g, masked):
                e = enc_v[pl.ds(g * LANES, LANES)]
                loc = jnp.bitwise_and(e, MASK14)
                p = jnp.bitwise_and(
                    lax.shift_right_logical(e, 14), MASK14
                )
                v0 = plsc.load_gather(cur, [row0, loc])
                v1 = plsc.load_gather(cur, [row1, loc])
                if masked:
                    k = g * LANES + lane
                    m = jnp.logical_and(k >= s_slot, k < e_slot)
                    plsc.store_scatter(out_v, [row0, p], v0, mask=m)
                    plsc.store_scatter(out_v, [row1, p], v1, mask=m)
                else:
                    plsc.store_scatter(out_v, [row0, p], v0)
                    plsc.store_scatter(out_v, [row1, p], v1)

            @pl.when(gs < ge)
            def _():
                do_group(gs, True)

            @pl.when(ge > gs + 1)
            def _():
                do_group(ge - 1, True)

            @pl.loop(gs + 1, ge - 1)
            def _(g):
                do_group(g, False)

        h0 = pltpu.async_copy(out_v.at[0], ot_hbm.at[j0], sem_a)
        h1 = pltpu.async_copy(out_v.at[1], ot_hbm.at[j0 + 1], sem_b)
        h0.wait()
        h1.wait()

    ot = gather_kernel(wt, wtail, enc, group_bounds)
    return ot.T


# speedup vs baseline: 2.8974x; 1.0068x over previous
import jax
import jax.numpy as jnp
from jax import lax
from jax.experimental import pallas as pl
from jax.experimental.pallas import tpu as pltpu
from jax.experimental.pallas import tpu_sc as plsc

LANES = 16
NUM_WORKERS = 32
PANEL_W = 16384
MASK14 = PANEL_W - 1


def kernel(x, weight):
    batch = x.shape[0]
    num_rows, embed_dim = weight.shape
    n_buckets = (num_rows + PANEL_W - 1) // PANEL_W
    n_groups = batch // LANES

    wt = weight.T

    xu = x.astype(jnp.uint32)
    pos = jnp.arange(batch, dtype=jnp.uint32)
    key = (
        lax.shift_left(lax.shift_right_logical(xu, jnp.uint32(14)), jnp.uint32(28))
        | lax.shift_left(pos, jnp.uint32(14))
        | (xu & jnp.uint32(MASK14))
    )
    key = jnp.sort(key)
    enc = lax.bitcast_convert_type(key, jnp.int32)
    b = lax.shift_right_logical(x, 14)
    cnt = jnp.sum(
        (b[:, None] == jnp.arange(n_buckets, dtype=jnp.int32)).astype(
            jnp.int32
        ),
        axis=0,
    )
    ends = jnp.cumsum(cnt)
    starts = ends - cnt
    group_bounds = jnp.concatenate([starts, ends])

    mesh = plsc.VectorSubcoreMesh(
        core_axis_name="core", subcore_axis_name="subcore"
    )
    last_start = (n_buckets - 1) * PANEL_W
    last_full = ((num_rows - last_start) // 128) * 128
    tail_w = num_rows - last_start - last_full
    wtail = jnp.pad(
        weight[num_rows - tail_w :].T, ((0, 0), (0, 128 - tail_w))
    )

    @pl.kernel(
        out_type=jax.ShapeDtypeStruct((embed_dim, batch), weight.dtype),
        mesh=mesh,
        compiler_params=pltpu.CompilerParams(needs_layout_passes=False),
        scratch_types=[
            pltpu.VMEM((2, PANEL_W), jnp.float32),
            pltpu.VMEM((2, PANEL_W), jnp.float32),
            pltpu.VMEM((2, batch), jnp.float32),
            pltpu.VMEM((batch,), jnp.int32),
            pltpu.VMEM((2 * n_buckets,), jnp.int32),
            pltpu.SemaphoreType.DMA,
            pltpu.SemaphoreType.DMA,
            pltpu.SemaphoreType.DMA,
            pltpu.SemaphoreType.DMA,
        ],
    )
    def gather_kernel(
        wt_hbm, wtail_hbm, enc_hbm, bounds_hbm, ot_hbm,
        panel_a, panel_b, out_v, enc_v, bounds_v,
        sem_e, sem_s, sem_a, sem_b,
    ):
        w = lax.axis_index("subcore") * 2 + lax.axis_index("core")
        j0 = 2 * w

        h_enc = pltpu.async_copy(enc_hbm, enc_v, sem_e)
        pltpu.sync_copy(bounds_hbm, bounds_v)
        lane = lax.iota(jnp.int32, LANES)
        starts_v = bounds_v[pl.ds(0, LANES)]
        ends_v = bounds_v[pl.ds(n_buckets, LANES)]

        bufs = (panel_a, panel_b)
        sems = (sem_a, sem_b)

        def issue_panel(bk, buf, sem):
            start = bk * PANEL_W
            half = PANEL_W // 2
            if bk < n_buckets - 1:
                return [
                    pltpu.async_copy(
                        wt_hbm.at[pl.ds(j0, 2), pl.ds(start, half)],
                        buf.at[:, pl.ds(0, half)],
                        sem,
                    ),
                    pltpu.async_copy(
                        wt_hbm.at[pl.ds(j0, 2), pl.ds(start + half, half)],
                        buf.at[:, pl.ds(half, half)],
                        sem,
                    ),
                ]
            hs = [
                pltpu.async_copy(
                    wt_hbm.at[pl.ds(j0, 2), pl.ds(start, last_full)],
                    buf.at[:, pl.ds(0, last_full)],
                    sem,
                )
            ]
            if tail_w:
                hs.append(
                    pltpu.async_copy(
                        wtail_hbm.at[pl.ds(j0, 2), :],
                        buf.at[:, pl.ds(last_full, 128)],
                        sem,
                    )
                )
            return hs

        handles = [issue_panel(0, bufs[0], sems[0]), None]
        h_enc.wait()

        row0 = jnp.zeros((LANES,), jnp.int32)
        row1 = jnp.ones((LANES,), jnp.int32)

        for bk in range(n_buckets):
            cur = bufs[bk & 1]
            for h in handles[bk & 1]:
                h.wait()
            if bk < n_buckets - 1:
                handles[(bk + 1) & 1] = issue_panel(
                    bk + 1, bufs[(bk + 1) & 1], sems[(bk + 1) & 1]
                )
            sel = (lane == bk).astype(jnp.int32)
            s_slot = jnp.sum(sel * starts_v, axis=0)
            e_slot = jnp.sum(sel * ends_v, axis=0)
            gs = lax.shift_right_logical(s_slot, 4)
            ge = lax.shift_right_logical(e_slot + LANES - 1, 4)

            def do_group(g, masked):
                e = enc_v[pl.ds(g * LANES, LANES)]
                loc = jnp.bitwise_and(e, MASK14)
                p = jnp.bitwise_and(
                    lax.shift_right_logical(e, 14), MASK14
                )
                v0 = plsc.load_gather(cur, [row0, loc])
                v1 = plsc.load_gather(cur, [row1, loc])
                if masked:
                    k = g * LANES + lane
                    m = jnp.logical_and(k >= s_slot, k < e_slot)
                    plsc.store_scatter(out_v, [row0, p], v0, mask=m)
                    plsc.store_scatter(out_v, [row1, p], v1, mask=m)
                else:
                    plsc.store_scatter(out_v, [row0, p], v0)
                    plsc.store_scatter(out_v, [row1, p], v1)

            @pl.when(gs < ge)
            def _():
                do_group(gs, True)

            @pl.when(ge > gs + 1)
            def _():
                do_group(ge - 1, True)

            @pl.loop(gs + 1, ge - 1)
            def _(g):
                do_group(g, False)

        h0 = pltpu.async_copy(out_v.at[0], ot_hbm.at[j0], sem_a)
        h1 = pltpu.async_copy(out_v.at[1], ot_hbm.at[j0 + 1], sem_b)
        h0.wait()
        h1.wait()

    ot = gather_kernel(wt, wtail, enc, group_bounds)
    return ot.T
